# baseline (speedup 1.0000x reference)
import jax
import jax.numpy as jnp
from jax.experimental import pallas as pl

_RATIO = 16


def _body(mask_ref, w_ref, emb_ref, gmask_ref):
    B, Sgb, D = emb_ref.shape
    w0 = w_ref[0, :]
    emb_ref[...] = jnp.broadcast_to(w0[None, None, :], (B, Sgb, D))

    @pl.when(pl.program_id(0) == 0)
    def _():
        emb_ref[:, 0, :] = jnp.broadcast_to(w_ref[1, :][None, :], (B, D))

    gmask_ref[...] = jnp.all(mask_ref[...], axis=2)


def kernel(token_ids, padding_mask, embeds_weight):
    B, Sl = padding_mask.shape
    Sg = Sl // _RATIO
    D = embeds_weight.shape[1]
    mask3 = padding_mask.reshape(B, Sg, _RATIO)
    sgb = 128
    grid = (Sg // sgb,)
    emb, gmask = pl.pallas_call(
        _body,
        grid=grid,
        in_specs=[
            pl.BlockSpec((B, sgb, _RATIO), lambda i: (0, i, 0)),
            pl.BlockSpec((2, D), lambda i: (0, 0)),
        ],
        out_specs=(
            pl.BlockSpec((B, sgb, D), lambda i: (0, i, 0)),
            pl.BlockSpec((B, sgb), lambda i: (0, i)),
        ),
        out_shape=(
            jax.ShapeDtypeStruct((B, Sg, D), embeds_weight.dtype),
            jax.ShapeDtypeStruct((B, Sg), jnp.bool_),
        ),
    )(mask3, embeds_weight)
    return (emb, gmask)
